# Initial kernel scaffold; baseline (speedup 1.0000x reference)
import jax
import jax.numpy as jnp
from jax.experimental import pallas as pl


def kernel(x, edge_index, edge_attr, Wp, bp, We1, be1, We2, be2, W1, b1, W2, b2, W3, b3, g1, bt1, g2, bt2, g3, bt3, attn):
    raise NotImplementedError("write your pallas kernel here")


# speedup vs baseline: 2.3981x; 2.3981x over previous
import jax
import jax.numpy as jnp
from jax import lax
from jax.experimental import pallas as pl
from jax.experimental.pallas import tpu as pltpu
from jax.experimental.pallas import tpu_sc as plsc

N = 100000
D = 128
L = 16
NSLAB = D // L
NC, NS = 2, 16
NW = NC * NS
EPAD = 102400
NPAD = 102400
CH = 64
WCH = EPAD // NW // CH
SCH = EPAD // NS // CH
K = 5
G = SCH // K
STRIPE = NPAD // NS
ZROWS = 160
BN = 2048
GRID = (N + BN - 1) // BN

_MESH = plsc.VectorSubcoreMesh(core_axis_name="c", subcore_axis_name="s")
_SC_PARAMS = pltpu.CompilerParams(use_tc_tiling_on_sc=False)


def _zero_fill(zbuf):
    def zf(r, carry):
        zbuf[r] = jnp.zeros((L,), jnp.float32)
        return carry
    lax.fori_loop(0, ZROWS, zf, 0)


def _zero_stripe(zbuf, shared, s):
    for k in range(STRIPE // ZROWS):
        pltpu.sync_copy(zbuf, shared.at[pl.ds(s * STRIPE + k * ZROWS, ZROWS)])


def _load_gidx(src64, dstS, gidx, s, c, dst64=None):
    pltpu.sync_copy(src64.at[2 * s], gidx.at[pl.ds(0, WCH)])
    pltpu.sync_copy(src64.at[2 * s + 1], gidx.at[pl.ds(WCH, WCH)])
    if dst64 is not None:
        pltpu.sync_copy(dst64.at[2 * s], dstS.at[pl.ds(0, WCH)])
        pltpu.sync_copy(dst64.at[2 * s + 1], dstS.at[pl.ds(WCH, WCH)])
    slab0 = c * (NSLAB // NC)

    def gx(j, carry):
        for i in range(CH // L):
            gidx[j, pl.ds(i * L, L)] = gidx[j, pl.ds(i * L, L)] * NSLAB + slab0
        return carry
    lax.fori_loop(0, SCH, gx, 0)


def _bump_gidx(gidx):
    def gx(j, carry):
        for i in range(CH // L):
            gidx[j, pl.ds(i * L, L)] = gidx[j, pl.ds(i * L, L)] + 1
        return carry
    lax.fori_loop(0, SCH, gx, 0)


def _pipe_chunks(hsv, gidx, rows, gsems, osems, emit, drain_emit):
    def issue_gathers(st, base):
        for b in range(K):
            pltpu.async_copy(hsv.at[gidx.at[base + b]], rows.at[st, b],
                             gsems[st])

    def drain_gathers(st):
        for b in range(K):
            pltpu.make_async_copy(hsv.at[pl.ds(0, CH)], rows.at[st, b],
                                  gsems[st]).wait()

    def drain_emits(st):
        for b in range(K):
            drain_emit(st, b)

    def body(g, carry):
        base = g * K
        for st in (0, 1):
            @pl.when((g % 2) == st)
            def _(st=st, base=base, g=g):
                @pl.when(g >= 2)
                def _():
                    drain_emits(st)
                issue_gathers(st, base)

                @pl.when(g >= 1)
                def _():
                    drain_gathers(1 - st)
                    for b in range(K):
                        emit(1 - st, b, base - K)
        return carry
    lax.fori_loop(0, G, body, 0)
    drain_gathers(1)
    for b in range(K):
        emit(1, b, (G - 1) * K)
    drain_emits(0)
    drain_emits(1)


def _make_sc_pre():
    def body(dst64, src64, ones_hbm, efv, deg_out, esrc_out,
             dstbuf, ones_v, zbuf, gidx, rows, shared,
             gsem0, gsem1, wsem0, wsem1, dsem):
        c = lax.axis_index("c")
        s = lax.axis_index("s")
        _zero_fill(zbuf)
        _zero_stripe(zbuf, shared, s)
        pltpu.sync_copy(ones_hbm, ones_v)
        pltpu.sync_copy(dst64.at[c * NS + s], dstbuf)
        _load_gidx(src64, None, gidx, s, c)
        plsc.subcore_barrier()

        def dadd(j, carry):
            pltpu.async_copy(ones_v, shared.at[dstbuf.at[j]], dsem, add=True)
            return carry
        lax.fori_loop(0, WCH, dadd, 0)

        gsems = (gsem0, gsem1)
        wsems = (wsem0, wsem1)

        def slab_iter(si, carry):
            slab = c * (NSLAB // NC) + si

            @pl.when(si > 0)
            def _():
                _bump_gidx(gidx)

            def emit(st, b, base):
                pltpu.async_copy(
                    rows.at[st, b],
                    esrc_out.at[slab, pl.ds(s * (SCH * CH) + (base + b) * CH,
                                            CH)],
                    wsems[st])

            def drain_emit(st, b):
                pltpu.make_async_copy(efv.at[pl.ds(0, CH)], rows.at[st, b],
                                      wsems[st]).wait()

            _pipe_chunks(efv, gidx, rows, gsems, wsems, emit, drain_emit)
            return carry
        lax.fori_loop(0, NSLAB // NC, slab_iter, 0)

        def ddrain(j, carry):
            pltpu.make_async_copy(efv.at[pl.ds(0, CH)], ones_v, dsem).wait()
            return carry
        lax.fori_loop(0, WCH, ddrain, 0)
        plsc.subcore_barrier()
        pltpu.sync_copy(shared.at[pl.ds(s * STRIPE, STRIPE)],
                        deg_out.at[c, pl.ds(s * STRIPE, STRIPE)])

    return pl.kernel(
        body,
        out_type=(jax.ShapeDtypeStruct((NC, NPAD, L), jnp.float32),
                  jax.ShapeDtypeStruct((NSLAB, EPAD, L), jnp.float32)),
        mesh=_MESH,
        compiler_params=_SC_PARAMS,
        scratch_types=[
            pltpu.VMEM((WCH, CH), jnp.int32),
            pltpu.VMEM((CH, L), jnp.float32),
            pltpu.VMEM((ZROWS, L), jnp.float32),
            pltpu.VMEM((SCH, CH), jnp.int32),
            pltpu.VMEM((2, K, CH, L), jnp.float32),
            pltpu.VMEM_SHARED((NPAD, L), jnp.float32),
            pltpu.SemaphoreType.DMA,
            pltpu.SemaphoreType.DMA,
            pltpu.SemaphoreType.DMA,
            pltpu.SemaphoreType.DMA,
            pltpu.SemaphoreType.DMA,
        ],
    )


def _make_sc_agg():
    def body(src64, dst64, hsv, agg_out, dstS, zbuf, gidx, rows, shared,
             gsem0, gsem1, ssem0, ssem1):
        c = lax.axis_index("c")
        s = lax.axis_index("s")
        _zero_fill(zbuf)
        _load_gidx(src64, dstS, gidx, s, c, dst64=dst64)
        gsems = (gsem0, gsem1)
        ssems = (ssem0, ssem1)

        def slab_iter(si, carry):
            slab = c * (NSLAB // NC) + si
            _zero_stripe(zbuf, shared, s)

            @pl.when(si > 0)
            def _():
                _bump_gidx(gidx)
            plsc.subcore_barrier()

            def emit(st, b, base):
                pltpu.async_copy(rows.at[st, b], shared.at[dstS.at[base + b]],
                                 ssems[st], add=True)

            def drain_emit(st, b):
                pltpu.make_async_copy(hsv.at[pl.ds(0, CH)], rows.at[st, b],
                                      ssems[st]).wait()

            _pipe_chunks(hsv, gidx, rows, gsems, ssems, emit, drain_emit)
            plsc.subcore_barrier()
            pltpu.sync_copy(shared.at[pl.ds(s * STRIPE, STRIPE)],
                            agg_out.at[slab, pl.ds(s * STRIPE, STRIPE)])
            return carry
        lax.fori_loop(0, NSLAB // NC, slab_iter, 0)

    return pl.kernel(
        body,
        out_type=jax.ShapeDtypeStruct((NSLAB, NPAD, L), jnp.float32),
        mesh=_MESH,
        compiler_params=_SC_PARAMS,
        scratch_types=[
            pltpu.VMEM((SCH, CH), jnp.int32),
            pltpu.VMEM((ZROWS, L), jnp.float32),
            pltpu.VMEM((SCH, CH), jnp.int32),
            pltpu.VMEM((2, K, CH, L), jnp.float32),
            pltpu.VMEM_SHARED((NPAD, L), jnp.float32),
            pltpu.SemaphoreType.DMA,
            pltpu.SemaphoreType.DMA,
            pltpu.SemaphoreType.DMA,
            pltpu.SemaphoreType.DMA,
        ],
    )


_sc_pre = _make_sc_pre()
_sc_agg = _make_sc_agg()


def _ln(t, g, b):
    m = jnp.mean(t, axis=-1, keepdims=True)
    v = jnp.mean((t - m) * (t - m), axis=-1, keepdims=True)
    return (t - m) * lax.rsqrt(v + 1e-5) * g + b


def _dot(a, b):
    return jnp.dot(a, b, preferred_element_type=jnp.float32)


def _cat(agg_ref):
    return jnp.concatenate([agg_ref[k] for k in range(NSLAB)], axis=-1)


_ROWB = lambda: pl.BlockSpec((BN, D), lambda i: (i, 0))
_ROW1 = lambda: pl.BlockSpec((BN, 1), lambda i: (i, 0))
_AGGB = lambda: pl.BlockSpec((NSLAB, BN, L), lambda i: (0, i, 0))
_FULL = lambda r, c: pl.BlockSpec((r, c), lambda i: (0, 0))


def _tc0_body(ea_ref, We1_ref, be1_ref, We2_ref, be2_ref, ef_ref):
    e = jnp.maximum(_dot(ea_ref[...], We1_ref[...]) + be1_ref[...], 0.0)
    ef_ref[...] = _dot(e, We2_ref[...]) + be2_ref[...]


def _tc0(ea, We1, be1, We2, be2):
    return pl.pallas_call(
        _tc0_body,
        grid=(GRID,),
        in_specs=[
            pl.BlockSpec((BN, 16), lambda i: (i, 0)),
            _FULL(16, D), _FULL(1, D), _FULL(D, D), _FULL(1, D),
        ],
        out_specs=_ROWB(),
        out_shape=jax.ShapeDtypeStruct((N, D), jnp.float32),
    )(ea, We1, be1, We2, be2)


def _tc1_body(x_ref, deg_ref, Wp_ref, bp_ref, W1_ref, hs1_ref, dis_ref):
    d = deg_ref[0, :, 0:1] + deg_ref[1, :, 0:1] + 1.0
    dis = lax.rsqrt(d)
    h0 = _dot(x_ref[...], Wp_ref[...]) + bp_ref[...]
    hs1_ref[...] = _dot(h0, W1_ref[...]) * dis
    dis_ref[...] = dis


def _tc1(x, deg16, Wp, bp, W1):
    return pl.pallas_call(
        _tc1_body,
        grid=(GRID,),
        in_specs=[
            _ROWB(),
            pl.BlockSpec((NC, BN, L), lambda i: (0, i, 0)),
            _FULL(D, D), _FULL(1, D), _FULL(D, D),
        ],
        out_specs=[_ROWB(), _ROW1()],
        out_shape=[jax.ShapeDtypeStruct((N, D), jnp.float32),
                   jax.ShapeDtypeStruct((N, 1), jnp.float32)],
    )(x, deg16, Wp, bp, W1)


def _tc2_body(x_ref, agg_ref, esrc_ref, dis_ref, Wp_ref, bp_ref, W1_ref,
              W2_ref, b1_ref, g1_ref, bt1_ref, x1_ref, hs2_ref):
    dis = dis_ref[...]
    h0 = _dot(x_ref[...], Wp_ref[...]) + bp_ref[...]
    hs1 = _dot(h0, W1_ref[...]) * dis
    t = dis * (_cat(agg_ref) + hs1) + b1_ref[...] + _cat(esrc_ref)
    t = _ln(t, g1_ref[...], bt1_ref[...])
    x1 = jnp.maximum(t + h0, 0.0)
    x1_ref[...] = x1
    hs2_ref[...] = _dot(x1, W2_ref[...]) * dis


def _tc2(x, agg1, esrc, dis, Wp, bp, W1, W2, b1, g1, bt1):
    return pl.pallas_call(
        _tc2_body,
        grid=(GRID,),
        in_specs=[
            _ROWB(), _AGGB(), _AGGB(), _ROW1(),
            _FULL(D, D), _FULL(1, D), _FULL(D, D), _FULL(D, D),
            _FULL(1, D), _FULL(1, D), _FULL(1, D),
        ],
        out_specs=[_ROWB(), _ROWB()],
        out_shape=[jax.ShapeDtypeStruct((N, D), jnp.float32),
                   jax.ShapeDtypeStruct((N, D), jnp.float32)],
    )(x, agg1, esrc, dis, Wp, bp, W1, W2, b1, g1, bt1)


def _tc3_body(agg_ref, hs2_ref, x1_ref, esrc_ref, dis_ref, W3_ref, b2_ref,
              g2_ref, bt2_ref, hs3_ref):
    dis = dis_ref[...]
    t = dis * (_cat(agg_ref) + hs2_ref[...]) + b2_ref[...] + _cat(esrc_ref)
    t = _ln(t, g2_ref[...], bt2_ref[...])
    x2 = jnp.maximum(t + x1_ref[...], 0.0)
    hs3_ref[...] = _dot(x2, W3_ref[...]) * dis


def _tc3(agg2, hs2, x1, esrc, dis, W3, b2, g2, bt2):
    return pl.pallas_call(
        _tc3_body,
        grid=(GRID,),
        in_specs=[
            _AGGB(), _ROWB(), _ROWB(), _AGGB(), _ROW1(),
            _FULL(D, D), _FULL(1, D), _FULL(1, D), _FULL(1, D),
        ],
        out_specs=_ROWB(),
        out_shape=jax.ShapeDtypeStruct((N, D), jnp.float32),
    )(agg2, hs2, x1, esrc, dis, W3, b2, g2, bt2)


def _tc4_body(agg_ref, hs3_ref, esrc_ref, dis_ref, b3_ref, g3_ref, bt3_ref,
              attn_ref, x3_ref, log_ref):
    dis = dis_ref[...]
    t = dis * (_cat(agg_ref) + hs3_ref[...]) + b3_ref[...] + _cat(esrc_ref)
    x3 = _ln(t, g3_ref[...], bt3_ref[...])
    x3_ref[...] = x3
    log_ref[...] = _dot(x3, attn_ref[...])


def _tc4(agg3, hs3, esrc, dis, b3, g3, bt3, attn):
    return pl.pallas_call(
        _tc4_body,
        grid=(GRID,),
        in_specs=[
            _AGGB(), _ROWB(), _AGGB(), _ROW1(),
            _FULL(1, D), _FULL(1, D), _FULL(1, D), _FULL(D, 1),
        ],
        out_specs=[_ROWB(), _ROW1()],
        out_shape=[jax.ShapeDtypeStruct((N, D), jnp.float32),
                   jax.ShapeDtypeStruct((N, 1), jnp.float32)],
    )(agg3, hs3, esrc, dis, b3, g3, bt3, attn)


def _stats_body(log_ref, m_ref, s_ref, m_sc, s_sc):
    i = pl.program_id(0)

    @pl.when(i == 0)
    def _():
        m_sc[0] = -jnp.inf
        s_sc[0] = 0.0

    blk = log_ref[...]
    rows = i * BN + lax.broadcasted_iota(jnp.int32, (BN, 1), 0)
    lv = jnp.where(rows < N, blk, -jnp.inf)
    m_old = m_sc[0]
    m_new = jnp.maximum(m_old, jnp.max(lv))
    s_new = s_sc[0] * jnp.exp(m_old - m_new) + jnp.sum(jnp.exp(lv - m_new))
    m_sc[0] = m_new
    s_sc[0] = s_new

    @pl.when(i == GRID - 1)
    def _():
        m_ref[0, 0] = m_new
        s_ref[0, 0] = s_new


def _stats(logits):
    return pl.pallas_call(
        _stats_body,
        grid=(GRID,),
        in_specs=[_ROW1()],
        out_specs=[pl.BlockSpec(memory_space=pltpu.SMEM),
                   pl.BlockSpec(memory_space=pltpu.SMEM)],
        out_shape=[jax.ShapeDtypeStruct((1, 1), jnp.float32),
                   jax.ShapeDtypeStruct((1, 1), jnp.float32)],
        scratch_shapes=[pltpu.SMEM((1,), jnp.float32),
                        pltpu.SMEM((1,), jnp.float32)],
    )(logits)


def _fin_body(x3_ref, log_ref, m_ref, s_ref, out_ref):
    w = jnp.exp(log_ref[...] - m_ref[0, 0]) / s_ref[0, 0]
    out_ref[...] = x3_ref[...] * w


def _fin(x3, logits, m, s):
    return pl.pallas_call(
        _fin_body,
        grid=(GRID,),
        in_specs=[_ROWB(), _ROW1(),
                  pl.BlockSpec(memory_space=pltpu.SMEM),
                  pl.BlockSpec(memory_space=pltpu.SMEM)],
        out_specs=_ROWB(),
        out_shape=jax.ShapeDtypeStruct((N, D), jnp.float32),
    )(x3, logits, m, s)


def kernel(x, edge_index, edge_attr, Wp, bp, We1, be1, We2, be2, W1, b1,
           W2, b2, W3, b3, g1, bt1, g2, bt2, g3, bt3, attn):
    src = edge_index[0].astype(jnp.int32)
    dst = edge_index[1].astype(jnp.int32)
    pad = EPAD - src.shape[0]
    srcp = jnp.concatenate([src, jnp.zeros((pad,), jnp.int32)])
    dstp = jnp.concatenate([dst, jnp.full((pad,), N, jnp.int32)])
    src64 = srcp.reshape(NW, WCH, CH)
    dst64 = dstp.reshape(NW, WCH, CH)
    ones_sc = jnp.ones((CH, L), jnp.float32)
    row = lambda v: v.reshape(1, D)

    ef = _tc0(edge_attr, We1, row(be1), We2, row(be2))
    deg16, esrc = _sc_pre(dst64, src64, ones_sc, ef.reshape(N * NSLAB, L))
    hs1, dis = _tc1(x, deg16, Wp, row(bp), W1)
    agg1 = _sc_agg(src64, dst64, hs1.reshape(N * NSLAB, L))
    x1, hs2 = _tc2(x, agg1, esrc, dis, Wp, row(bp), W1, W2, row(b1),
                   row(g1), row(bt1))
    agg2 = _sc_agg(src64, dst64, hs2.reshape(N * NSLAB, L))
    hs3 = _tc3(agg2, hs2, x1, esrc, dis, W3, row(b2), row(g2), row(bt2))
    agg3 = _sc_agg(src64, dst64, hs3.reshape(N * NSLAB, L))
    x3, logits = _tc4(agg3, hs3, esrc, dis, row(b3), row(g3), row(bt3),
                      attn.reshape(D, 1))
    m, s = _stats(logits)
    return _fin(x3, logits, m, s)


# speedup vs baseline: 3.9932x; 1.6651x over previous
import jax
import jax.numpy as jnp
from jax import lax
from jax.experimental import pallas as pl
from jax.experimental.pallas import tpu as pltpu
from jax.experimental.pallas import tpu_sc as plsc

N = 100000
D = 128
L = 16
NSLAB = D // L
NC, NS = 2, 16
NW = NC * NS
EPAD = 102400
NPAD = 102400
CH = 128
WCH = EPAD // NW // CH
SCH = EPAD // NS // CH
K = 3
G = SCH // K
GR = SCH - G * K
STRIPE = NPAD // NS
ZROWS = 160
BN = 2048
GRID = (N + BN - 1) // BN

_MESH = plsc.VectorSubcoreMesh(core_axis_name="c", subcore_axis_name="s")
_SC_PARAMS = pltpu.CompilerParams(use_tc_tiling_on_sc=False)


def _zero_fill(zbuf):
    def zf(r, carry):
        zbuf[r] = jnp.zeros((L,), jnp.float32)
        return carry
    lax.fori_loop(0, ZROWS, zf, 0)


def _zero_stripe(zbuf, shared, s):
    for k in range(STRIPE // ZROWS):
        pltpu.sync_copy(zbuf, shared.at[pl.ds(s * STRIPE + k * ZROWS, ZROWS)])


def _load_gidx(src64, dstS, gidx, s, c, dst64=None):
    pltpu.sync_copy(src64.at[2 * s], gidx.at[pl.ds(0, WCH)])
    pltpu.sync_copy(src64.at[2 * s + 1], gidx.at[pl.ds(WCH, WCH)])
    if dst64 is not None:
        pltpu.sync_copy(dst64.at[2 * s], dstS.at[pl.ds(0, WCH)])
        pltpu.sync_copy(dst64.at[2 * s + 1], dstS.at[pl.ds(WCH, WCH)])
    slab0 = c * (NSLAB // NC)

    def gx(j, carry):
        for i in range(CH // L):
            gidx[j, pl.ds(i * L, L)] = gidx[j, pl.ds(i * L, L)] * NSLAB + slab0
        return carry
    lax.fori_loop(0, SCH, gx, 0)


def _bump_gidx(gidx):
    def gx(j, carry):
        for i in range(CH // L):
            gidx[j, pl.ds(i * L, L)] = gidx[j, pl.ds(i * L, L)] + 1
        return carry
    lax.fori_loop(0, SCH, gx, 0)


def _pipe_chunks(hsv, gidx, rows, gsems, osems, emit, drain_emit):
    def issue_gathers(st, base):
        for b in range(K):
            pltpu.async_copy(hsv.at[gidx.at[base + b]], rows.at[st, b],
                             gsems[st])

    def drain_gathers(st):
        for b in range(K):
            pltpu.make_async_copy(hsv.at[pl.ds(0, CH)], rows.at[st, b],
                                  gsems[st]).wait()

    def drain_emits(st):
        for b in range(K):
            drain_emit(st, b)

    def body(g, carry):
        base = g * K
        for st in (0, 1):
            @pl.when((g % 2) == st)
            def _(st=st, base=base, g=g):
                @pl.when(g >= 2)
                def _():
                    drain_emits(st)
                issue_gathers(st, base)

                @pl.when(g >= 1)
                def _():
                    drain_gathers(1 - st)
                    for b in range(K):
                        emit(1 - st, b, base - K)
        return carry
    lax.fori_loop(0, G, body, 0)
    drain_gathers(1)
    for b in range(K):
        emit(1, b, (G - 1) * K)
    drain_emits(0)
    for b in range(GR):
        pltpu.async_copy(hsv.at[gidx.at[G * K + b]], rows.at[0, b], gsems[0])
    for b in range(GR):
        pltpu.make_async_copy(hsv.at[pl.ds(0, CH)], rows.at[0, b],
                              gsems[0]).wait()
    for b in range(GR):
        emit(0, b, G * K)
    for b in range(GR):
        drain_emit(0, b)
    drain_emits(1)


def _make_sc_deg():
    def body(dst64, ones_hbm, deg_out, dstbuf, ones_v, zbuf, shared, dsem):
        c = lax.axis_index("c")
        s = lax.axis_index("s")
        _zero_fill(zbuf)
        _zero_stripe(zbuf, shared, s)
        pltpu.sync_copy(ones_hbm, ones_v)
        pltpu.sync_copy(dst64.at[c * NS + s], dstbuf)
        plsc.subcore_barrier()

        def dadd(j, carry):
            pltpu.async_copy(ones_v, shared.at[dstbuf.at[j]], dsem, add=True)
            return carry
        lax.fori_loop(0, WCH, dadd, 0)

        def ddrain(j, carry):
            pltpu.make_async_copy(ones_hbm, ones_v, dsem).wait()
            return carry
        lax.fori_loop(0, WCH, ddrain, 0)
        plsc.subcore_barrier()
        pltpu.sync_copy(shared.at[pl.ds(s * STRIPE, STRIPE)],
                        deg_out.at[c, pl.ds(s * STRIPE, STRIPE)])

    return pl.kernel(
        body,
        out_type=jax.ShapeDtypeStruct((NC, NPAD, L), jnp.float32),
        mesh=_MESH,
        compiler_params=_SC_PARAMS,
        scratch_types=[
            pltpu.VMEM((WCH, CH), jnp.int32),
            pltpu.VMEM((CH, L), jnp.float32),
            pltpu.VMEM((ZROWS, L), jnp.float32),
            pltpu.VMEM_SHARED((NPAD, L), jnp.float32),
            pltpu.SemaphoreType.DMA,
        ],
    )


def _make_sc_esrc():
    def body(src64, efv, esrc_out, gidx, rows,
             gsem0, gsem1, wsem0, wsem1):
        c = lax.axis_index("c")
        s = lax.axis_index("s")
        _load_gidx(src64, None, gidx, s, c)
        gsems = (gsem0, gsem1)
        wsems = (wsem0, wsem1)

        def slab_iter(si, carry):
            slab = c * (NSLAB // NC) + si

            @pl.when(si > 0)
            def _():
                _bump_gidx(gidx)

            def emit(st, b, base):
                pltpu.async_copy(
                    rows.at[st, b],
                    esrc_out.at[pl.ds(s * (SCH * CH) + (base + b) * CH, CH),
                                pl.ds(slab * L, L)],
                    wsems[st])

            def drain_emit(st, b):
                pltpu.make_async_copy(efv.at[pl.ds(0, CH)], rows.at[st, b],
                                      wsems[st]).wait()

            _pipe_chunks(efv, gidx, rows, gsems, wsems, emit, drain_emit)
            return carry
        lax.fori_loop(0, NSLAB // NC, slab_iter, 0)

    return pl.kernel(
        body,
        out_type=jax.ShapeDtypeStruct((EPAD, D), jnp.float32),
        mesh=_MESH,
        compiler_params=_SC_PARAMS,
        scratch_types=[
            pltpu.VMEM((SCH, CH), jnp.int32),
            pltpu.VMEM((2, K, CH, L), jnp.float32),
            pltpu.SemaphoreType.DMA,
            pltpu.SemaphoreType.DMA,
            pltpu.SemaphoreType.DMA,
            pltpu.SemaphoreType.DMA,
        ],
    )


def _make_sc_agg():
    def body(src64, dst64, hsv, agg_out, dstS, zbuf, gidx, rows, shared,
             gsem0, gsem1, ssem0, ssem1):
        c = lax.axis_index("c")
        s = lax.axis_index("s")
        _zero_fill(zbuf)
        _load_gidx(src64, dstS, gidx, s, c, dst64=dst64)
        gsems = (gsem0, gsem1)
        ssems = (ssem0, ssem1)

        def slab_iter(si, carry):
            slab = c * (NSLAB // NC) + si
            _zero_stripe(zbuf, shared, s)

            @pl.when(si > 0)
            def _():
                _bump_gidx(gidx)
            plsc.subcore_barrier()

            def emit(st, b, base):
                pltpu.async_copy(rows.at[st, b], shared.at[dstS.at[base + b]],
                                 ssems[st], add=True)

            def drain_emit(st, b):
                pltpu.make_async_copy(hsv.at[pl.ds(0, CH)], rows.at[st, b],
                                      ssems[st]).wait()

            _pipe_chunks(hsv, gidx, rows, gsems, ssems, emit, drain_emit)
            plsc.subcore_barrier()
            pltpu.sync_copy(shared.at[pl.ds(s * STRIPE, STRIPE)],
                            agg_out.at[pl.ds(s * STRIPE, STRIPE),
                                       pl.ds(slab * L, L)])
            return carry
        lax.fori_loop(0, NSLAB // NC, slab_iter, 0)

    return pl.kernel(
        body,
        out_type=jax.ShapeDtypeStruct((NPAD, D), jnp.float32),
        mesh=_MESH,
        compiler_params=_SC_PARAMS,
        scratch_types=[
            pltpu.VMEM((SCH, CH), jnp.int32),
            pltpu.VMEM((ZROWS, L), jnp.float32),
            pltpu.VMEM((SCH, CH), jnp.int32),
            pltpu.VMEM((2, K, CH, L), jnp.float32),
            pltpu.VMEM_SHARED((NPAD, L), jnp.float32),
            pltpu.SemaphoreType.DMA,
            pltpu.SemaphoreType.DMA,
            pltpu.SemaphoreType.DMA,
            pltpu.SemaphoreType.DMA,
        ],
    )


_sc_deg = _make_sc_deg()
_sc_esrc = _make_sc_esrc()
_sc_agg = _make_sc_agg()


def _ln(t, g, b):
    m = jnp.mean(t, axis=-1, keepdims=True)
    v = jnp.mean((t - m) * (t - m), axis=-1, keepdims=True)
    return (t - m) * lax.rsqrt(v + 1e-5) * g + b


def _dot(a, b):
    return jnp.dot(a, b, preferred_element_type=jnp.float32)


_ROWB = lambda: pl.BlockSpec((BN, D), lambda i: (i, 0))
_ROW1 = lambda: pl.BlockSpec((BN, 1), lambda i: (i, 0))
_FULL = lambda r, c: pl.BlockSpec((r, c), lambda i: (0, 0))


def _tc0_body(ea_ref, We1_ref, be1_ref, We2_ref, be2_ref, ef_ref):
    e = jnp.maximum(_dot(ea_ref[...], We1_ref[...]) + be1_ref[...], 0.0)
    ef_ref[...] = _dot(e, We2_ref[...]) + be2_ref[...]


def _tc0(ea, We1, be1, We2, be2):
    return pl.pallas_call(
        _tc0_body,
        grid=(GRID,),
        in_specs=[
            pl.BlockSpec((BN, 16), lambda i: (i, 0)),
            _FULL(16, D), _FULL(1, D), _FULL(D, D), _FULL(1, D),
        ],
        out_specs=_ROWB(),
        out_shape=jax.ShapeDtypeStruct((N, D), jnp.float32),
    )(ea, We1, be1, We2, be2)


def _tc1_body(x_ref, deg_ref, Wp_ref, bp_ref, W1_ref, hs1_ref, dis_ref):
    d = deg_ref[0, :, 0:1] + deg_ref[1, :, 0:1] + 1.0
    dis = lax.rsqrt(d)
    h0 = _dot(x_ref[...], Wp_ref[...]) + bp_ref[...]
    hs1_ref[...] = _dot(h0, W1_ref[...]) * dis
    dis_ref[...] = dis


def _tc1(x, deg16, Wp, bp, W1):
    return pl.pallas_call(
        _tc1_body,
        grid=(GRID,),
        in_specs=[
            _ROWB(),
            pl.BlockSpec((NC, BN, L), lambda i: (0, i, 0)),
            _FULL(D, D), _FULL(1, D), _FULL(D, D),
        ],
        out_specs=[_ROWB(), _ROW1()],
        out_shape=[jax.ShapeDtypeStruct((N, D), jnp.float32),
                   jax.ShapeDtypeStruct((N, 1), jnp.float32)],
    )(x, deg16, Wp, bp, W1)


def _tc2_body(x_ref, agg_ref, esrc_ref, dis_ref, Wp_ref, bp_ref, W1_ref,
              W2_ref, b1_ref, g1_ref, bt1_ref, x1_ref, hs2_ref):
    dis = dis_ref[...]
    h0 = _dot(x_ref[...], Wp_ref[...]) + bp_ref[...]
    hs1 = _dot(h0, W1_ref[...]) * dis
    t = dis * (agg_ref[...] + hs1) + b1_ref[...] + esrc_ref[...]
    t = _ln(t, g1_ref[...], bt1_ref[...])
    x1 = jnp.maximum(t + h0, 0.0)
    x1_ref[...] = x1
    hs2_ref[...] = _dot(x1, W2_ref[...]) * dis


def _tc2(x, agg1, esrc, dis, Wp, bp, W1, W2, b1, g1, bt1):
    return pl.pallas_call(
        _tc2_body,
        grid=(GRID,),
        in_specs=[
            _ROWB(), _ROWB(), _ROWB(), _ROW1(),
            _FULL(D, D), _FULL(1, D), _FULL(D, D), _FULL(D, D),
            _FULL(1, D), _FULL(1, D), _FULL(1, D),
        ],
        out_specs=[_ROWB(), _ROWB()],
        out_shape=[jax.ShapeDtypeStruct((N, D), jnp.float32),
                   jax.ShapeDtypeStruct((N, D), jnp.float32)],
    )(x, agg1, esrc, dis, Wp, bp, W1, W2, b1, g1, bt1)


def _tc3_body(agg_ref, hs2_ref, x1_ref, esrc_ref, dis_ref, W3_ref, b2_ref,
              g2_ref, bt2_ref, hs3_ref):
    dis = dis_ref[...]
    t = dis * (agg_ref[...] + hs2_ref[...]) + b2_ref[...] + esrc_ref[...]
    t = _ln(t, g2_ref[...], bt2_ref[...])
    x2 = jnp.maximum(t + x1_ref[...], 0.0)
    hs3_ref[...] = _dot(x2, W3_ref[...]) * dis


def _tc3(agg2, hs2, x1, esrc, dis, W3, b2, g2, bt2):
    return pl.pallas_call(
        _tc3_body,
        grid=(GRID,),
        in_specs=[
            _ROWB(), _ROWB(), _ROWB(), _ROWB(), _ROW1(),
            _FULL(D, D), _FULL(1, D), _FULL(1, D), _FULL(1, D),
        ],
        out_specs=_ROWB(),
        out_shape=jax.ShapeDtypeStruct((N, D), jnp.float32),
    )(agg2, hs2, x1, esrc, dis, W3, b2, g2, bt2)


def _tc4_body(agg_ref, hs3_ref, esrc_ref, dis_ref, b3_ref, g3_ref, bt3_ref,
              attn_ref, x3_ref, log_ref):
    dis = dis_ref[...]
    t = dis * (agg_ref[...] + hs3_ref[...]) + b3_ref[...] + esrc_ref[...]
    x3 = _ln(t, g3_ref[...], bt3_ref[...])
    x3_ref[...] = x3
    log_ref[...] = _dot(x3, attn_ref[...])


def _tc4(agg3, hs3, esrc, dis, b3, g3, bt3, attn):
    return pl.pallas_call(
        _tc4_body,
        grid=(GRID,),
        in_specs=[
            _ROWB(), _ROWB(), _ROWB(), _ROW1(),
            _FULL(1, D), _FULL(1, D), _FULL(1, D), _FULL(D, 1),
        ],
        out_specs=[_ROWB(), _ROW1()],
        out_shape=[jax.ShapeDtypeStruct((N, D), jnp.float32),
                   jax.ShapeDtypeStruct((N, 1), jnp.float32)],
    )(agg3, hs3, esrc, dis, b3, g3, bt3, attn)


def _stats_body(log_ref, m_ref, s_ref, m_sc, s_sc):
    i = pl.program_id(0)

    @pl.when(i == 0)
    def _():
        m_sc[0] = -jnp.inf
        s_sc[0] = 0.0

    blk = log_ref[...]
    rows = i * BN + lax.broadcasted_iota(jnp.int32, (BN, 1), 0)
    lv = jnp.where(rows < N, blk, -jnp.inf)
    m_old = m_sc[0]
    m_new = jnp.maximum(m_old, jnp.max(lv))
    s_new = s_sc[0] * jnp.exp(m_old - m_new) + jnp.sum(jnp.exp(lv - m_new))
    m_sc[0] = m_new
    s_sc[0] = s_new

    @pl.when(i == GRID - 1)
    def _():
        m_ref[0, 0] = m_new
        s_ref[0, 0] = s_new


def _stats(logits):
    return pl.pallas_call(
        _stats_body,
        grid=(GRID,),
        in_specs=[_ROW1()],
        out_specs=[pl.BlockSpec(memory_space=pltpu.SMEM),
                   pl.BlockSpec(memory_space=pltpu.SMEM)],
        out_shape=[jax.ShapeDtypeStruct((1, 1), jnp.float32),
                   jax.ShapeDtypeStruct((1, 1), jnp.float32)],
        scratch_shapes=[pltpu.SMEM((1,), jnp.float32),
                        pltpu.SMEM((1,), jnp.float32)],
    )(logits)


def _fin_body(x3_ref, log_ref, m_ref, s_ref, out_ref):
    w = jnp.exp(log_ref[...] - m_ref[0, 0]) / s_ref[0, 0]
    out_ref[...] = x3_ref[...] * w


def _fin(x3, logits, m, s):
    return pl.pallas_call(
        _fin_body,
        grid=(GRID,),
        in_specs=[_ROWB(), _ROW1(),
                  pl.BlockSpec(memory_space=pltpu.SMEM),
                  pl.BlockSpec(memory_space=pltpu.SMEM)],
        out_specs=_ROWB(),
        out_shape=jax.ShapeDtypeStruct((N, D), jnp.float32),
    )(x3, logits, m, s)


def kernel(x, edge_index, edge_attr, Wp, bp, We1, be1, We2, be2, W1, b1,
           W2, b2, W3, b3, g1, bt1, g2, bt2, g3, bt3, attn):
    src = edge_index[0].astype(jnp.int32)
    dst = edge_index[1].astype(jnp.int32)
    pad = EPAD - src.shape[0]
    srcp = jnp.concatenate([src, jnp.zeros((pad,), jnp.int32)])
    dstp = jnp.concatenate([dst, jnp.full((pad,), N, jnp.int32)])
    src64 = srcp.reshape(NW, WCH, CH)
    dst64 = dstp.reshape(NW, WCH, CH)
    ones_sc = jnp.ones((CH, L), jnp.float32)
    row = lambda v: v.reshape(1, D)

    deg16 = _sc_deg(dst64, ones_sc)
    ef = _tc0(edge_attr, We1, row(be1), We2, row(be2))
    esrc = _sc_esrc(src64, ef.reshape(N * NSLAB, L))
    hs1, dis = _tc1(x, deg16, Wp, row(bp), W1)
    agg1 = _sc_agg(src64, dst64, hs1.reshape(N * NSLAB, L))
    x1, hs2 = _tc2(x, agg1, esrc, dis, Wp, row(bp), W1, W2, row(b1),
                   row(g1), row(bt1))
    agg2 = _sc_agg(src64, dst64, hs2.reshape(N * NSLAB, L))
    hs3 = _tc3(agg2, hs2, x1, esrc, dis, W3, row(b2), row(g2), row(bt2))
    agg3 = _sc_agg(src64, dst64, hs3.reshape(N * NSLAB, L))
    x3, logits = _tc4(agg3, hs3, esrc, dis, row(b3), row(g3), row(bt3),
                      attn.reshape(D, 1))
    m, s = _stats(logits)
    return _fin(x3, logits, m, s)
